# baseline (speedup 1.0000x reference)
import functools

import jax
import jax.numpy as jnp
from jax import lax
from jax.experimental import pallas as pl
from jax.experimental.pallas import tpu as pltpu
from jax.experimental.pallas import tpu_sc as plsc

_NUM_WORKERS = 32
_IDX_MINOR = 128
_K = 8


def _gather_call(n_rows, d):
    chunk = _K * _IDX_MINOR
    rows_per_worker = n_rows // _NUM_WORKERS
    n_chunks = rows_per_worker // chunk
    idx_rows_per_worker = rows_per_worker // _IDX_MINOR

    mesh = plsc.VectorSubcoreMesh(core_axis_name="c", subcore_axis_name="s")

    @functools.partial(
        pl.kernel,
        mesh=mesh,
        out_type=jax.ShapeDtypeStruct((n_rows, d), jnp.float32),
        scratch_types=[
            pltpu.VMEM((_K, _IDX_MINOR), jnp.int32),
            pltpu.VMEM((chunk, d), jnp.float32),
            pltpu.SemaphoreType.DMA,
        ],
        compiler_params=pltpu.CompilerParams(use_tc_tiling_on_sc=False),
    )
    def gather_kernel(idx_hbm, table_hbm, out_hbm, idx_v, rows_v, sem):
        wid = lax.axis_index("s") * 2 + lax.axis_index("c")
        idx_row_base = wid * idx_rows_per_worker
        out_base = wid * rows_per_worker

        def body(i, carry):
            pltpu.sync_copy(idx_hbm.at[pl.ds(idx_row_base + i * _K, _K)], idx_v)
            copies = []
            for j in range(_K):
                copies.append(
                    pltpu.async_copy(
                        table_hbm.at[idx_v.at[j]],
                        rows_v.at[pl.ds(j * _IDX_MINOR, _IDX_MINOR)],
                        sem,
                    )
                )
            for c in copies:
                c.wait()
            pltpu.sync_copy(rows_v, out_hbm.at[pl.ds(out_base + i * chunk, chunk)])
            return carry

        lax.fori_loop(0, n_chunks, body, 0)

    return gather_kernel


def kernel(x, table):
    b, s = x.shape
    n = b * s
    d = table.shape[1]
    idx = x.reshape(n // _IDX_MINOR, _IDX_MINOR).astype(jnp.int32)
    out = _gather_call(n, d)(idx, table)
    return out.reshape(b, s, d)


# speedup vs baseline: 1.0148x; 1.0148x over previous
import functools

import jax
import jax.numpy as jnp
from jax import lax
from jax.experimental import pallas as pl
from jax.experimental.pallas import tpu as pltpu
from jax.experimental.pallas import tpu_sc as plsc

_NUM_WORKERS = 32
_IDX_MINOR = 128
_K = 5


def _gather_call(n_rows, d):
    chunk = _K * _IDX_MINOR
    rows_per_worker = n_rows // _NUM_WORKERS
    n_chunks = rows_per_worker // chunk
    idx_rows_per_worker = rows_per_worker // _IDX_MINOR

    mesh = plsc.VectorSubcoreMesh(core_axis_name="c", subcore_axis_name="s")

    @functools.partial(
        pl.kernel,
        mesh=mesh,
        out_type=jax.ShapeDtypeStruct((n_rows, d), jnp.float32),
        scratch_types=[
            pltpu.VMEM((idx_rows_per_worker, _IDX_MINOR), jnp.int32),
            pltpu.VMEM((chunk, d), jnp.float32),
            pltpu.VMEM((chunk, d), jnp.float32),
            pltpu.SemaphoreType.DMA,
            pltpu.SemaphoreType.DMA,
            pltpu.SemaphoreType.DMA,
            pltpu.SemaphoreType.DMA,
        ],
        compiler_params=pltpu.CompilerParams(use_tc_tiling_on_sc=False),
    )
    def gather_kernel(idx_hbm, table_hbm, out_hbm, idx_v, rows0, rows1,
                      gs0, gs1, ss0, ss1):
        wid = lax.axis_index("s") * 2 + lax.axis_index("c")
        idx_row_base = wid * idx_rows_per_worker
        out_base = wid * rows_per_worker
        row_bufs = (rows0, rows1)
        gsems = (gs0, gs1)
        ssems = (ss0, ss1)

        def fire_gather(ci, b):
            for j in range(_K):
                pltpu.async_copy(
                    table_hbm.at[idx_v.at[ci * _K + j]],
                    row_bufs[b].at[pl.ds(j * _IDX_MINOR, _IDX_MINOR)],
                    gsems[b],
                )

        def wait_gather(b):
            pltpu.make_async_copy(
                table_hbm.at[pl.ds(0, chunk)], row_bufs[b], gsems[b]
            ).wait()

        def fire_store(ci, b):
            pltpu.async_copy(
                row_bufs[b],
                out_hbm.at[pl.ds(out_base + ci * chunk, chunk)],
                ssems[b],
            )

        def wait_store(b):
            pltpu.make_async_copy(
                row_bufs[b], out_hbm.at[pl.ds(out_base, chunk)], ssems[b]
            ).wait()

        pltpu.sync_copy(idx_hbm.at[pl.ds(idx_row_base, idx_rows_per_worker)],
                        idx_v)
        fire_gather(0, 0)
        fire_gather(1, 1)

        def body(p, carry):
            i = 2 * p
            for b in range(2):
                wait_gather(b)
                fire_store(i + b, b)
            for b in range(2):
                wait_store(b)
                fire_gather(i + b + 2, b)
            return carry

        lax.fori_loop(0, (n_chunks - 2) // 2, body, 0)

        for b in range(2):
            wait_gather(b)
            fire_store(n_chunks - 2 + b, b)
        for b in range(2):
            wait_store(b)

    return gather_kernel


def kernel(x, table):
    b, s = x.shape
    n = b * s
    d = table.shape[1]
    idx = x.reshape(n // _IDX_MINOR, _IDX_MINOR).astype(jnp.int32)
    out = _gather_call(n, d)(idx, table)
    return out.reshape(b, s, d)
